# Initial kernel scaffold; baseline (speedup 1.0000x reference)
import jax
import jax.numpy as jnp
from jax.experimental import pallas as pl


def kernel(node_features, node_attributes, sph_harmonics, radial_basis, edge_index, W_pre, W1, W2, W3, W_post, W_c1, W_c2, W_c3, W_res, W_out):
    raise NotImplementedError("write your pallas kernel here")


# speedup vs baseline: 1.6163x; 1.6163x over previous
import functools
import math

import jax
import jax.numpy as jnp
from jax import lax
from jax.experimental import pallas as pl
from jax.experimental.pallas import tpu as pltpu
from jax.experimental.pallas import tpu_sc as plsc

NC = 2
NS = 16
NW = NC * NS
LANES = 16

def _z():
    return jnp.int32(0)


def _pre_body(nf_ref, wpre_ref, h_ref, *, scale):
    h_ref[...] = jnp.dot(nf_ref[...], wpre_ref[...],
                         preferred_element_type=jnp.float32) * scale


def _pre_linear(nf, w_pre):
    n, c = nf.shape
    bn = 1000
    scale = 1.0 / math.sqrt(c)
    return pl.pallas_call(
        functools.partial(_pre_body, scale=scale),
        grid=(n // bn,),
        in_specs=[
            pl.BlockSpec((bn, c), lambda i: (i, _z())),
            pl.BlockSpec((c, c), lambda i: (_z(), _z())),
        ],
        out_specs=pl.BlockSpec((bn, c), lambda i: (i, _z())),
        out_shape=jax.ShapeDtypeStruct((n, c), jnp.float32),
    )(nf, w_pre)


def _coeff_body(rb_ref, sph_ref, w1_ref, w2_ref, w3_ref, coeff_ref):
    w = jax.nn.silu(jnp.dot(rb_ref[...], w1_ref[...],
                            preferred_element_type=jnp.float32))
    w = jax.nn.silu(jnp.dot(w, w2_ref[...],
                            preferred_element_type=jnp.float32))
    w = jnp.dot(w, w3_ref[...], preferred_element_type=jnp.float32)
    coeff_ref[...] = w * sph_ref[...]


def _edge_coeff(rb, sph, w1, w2, w3):
    e, r = rb.shape
    h = w1.shape[1]
    c = w3.shape[1]
    be = 8192
    return pl.pallas_call(
        _coeff_body,
        grid=(e // be,),
        in_specs=[
            pl.BlockSpec((be, r), lambda i: (i, _z())),
            pl.BlockSpec((be, 1), lambda i: (i, _z())),
            pl.BlockSpec((r, h), lambda i: (_z(), _z())),
            pl.BlockSpec((h, h), lambda i: (_z(), _z())),
            pl.BlockSpec((h, c), lambda i: (_z(), _z())),
        ],
        out_specs=pl.BlockSpec((be, c), lambda i: (i, _z())),
        out_shape=jax.ShapeDtypeStruct((e, c), jnp.float32),
    )(rb, sph, w1, w2, w3)


def _sc_aggregate(h, coeff, src3d, dst3d, *, n, c, e_pad, k, grp):
    epw = e_pad // NW
    nchunk = epw // k
    ngrp = nchunk // grp
    zrows = 32
    n_pad = ((n + NS * zrows - 1) // (NS * zrows)) * (NS * zrows)
    rpt = n_pad // NS
    assert rpt % zrows == 0 and nchunk % grp == 0 and grp % 8 == 0

    mesh = plsc.VectorSubcoreMesh(core_axis_name="c", subcore_axis_name="s")

    @functools.partial(
        pl.kernel,
        out_type=jax.ShapeDtypeStruct((NC, n_pad, c), jnp.float32),
        mesh=mesh,
        scratch_types=[
            pltpu.VMEM((grp, k), jnp.int32),
            pltpu.VMEM((grp, k), jnp.int32),
            pltpu.VMEM((k, c), jnp.float32),
            pltpu.VMEM((k, c), jnp.float32),
            pltpu.VMEM((zrows, c), jnp.float32),
            pltpu.VMEM_SHARED((n_pad, c), jnp.float32),
            pltpu.SemaphoreType.DMA,
        ],
    )
    def body(h_hbm, coeff_hbm, src_hbm, dst_hbm, out_hbm,
             src_v, dst_v, gbuf, cbuf, zbuf, acc, sem):
        cid = lax.axis_index("c").astype(jnp.int32)
        sid = lax.axis_index("s").astype(jnp.int32)
        wid = cid * jnp.int32(NS) + sid
        zero = jnp.zeros((LANES,), jnp.float32)

        def zrow(i, carry):
            for q in range(c // LANES):
                zbuf[i, pl.ds(q * LANES, LANES)] = zero
            return carry
        lax.fori_loop(jnp.int32(0), jnp.int32(zrows), zrow, jnp.int32(0))
        for t in range(rpt // zrows):
            pltpu.sync_copy(zbuf, acc.at[pl.ds(sid * rpt + t * zrows, zrows)])
        plsc.subcore_barrier()

        cbase = wid * jnp.int32(epw)

        def group(g, carry):
            pltpu.sync_copy(src_hbm.at[wid, pl.ds(g * grp, grp)], src_v)
            pltpu.sync_copy(dst_hbm.at[wid, pl.ds(g * grp, grp)], dst_v)

            def chunk(jj, carry2):
                pltpu.async_copy(h_hbm.at[src_v.at[jj]], gbuf, sem).wait()
                pltpu.sync_copy(
                    coeff_hbm.at[pl.ds(cbase + (g * grp + jj) * jnp.int32(k), k)],
                    cbuf)

                def mrow(r2, carry3):
                    for q in range(c // LANES):
                        sl = pl.ds(q * LANES, LANES)
                        gbuf[r2, sl] = gbuf[r2, sl] * cbuf[r2, sl]
                    return carry3
                lax.fori_loop(jnp.int32(0), jnp.int32(k), mrow, jnp.int32(0))
                pltpu.sync_copy(gbuf, acc.at[dst_v.at[jj]], add=True)
                return carry2
            lax.fori_loop(jnp.int32(0), jnp.int32(grp), chunk, jnp.int32(0))
            return carry
        lax.fori_loop(jnp.int32(0), jnp.int32(ngrp), group, jnp.int32(0))

        plsc.subcore_barrier()
        pltpu.sync_copy(acc.at[pl.ds(sid * rpt, rpt)],
                        out_hbm.at[cid, pl.ds(sid * rpt, rpt)])

    return body(h, coeff, src3d, dst3d)


def _post_body(p_ref, nf_ref, attr_ref, wpost_ref, wc_ref, wres_ref,
               wout_ref, out_ref, *, a_dim, c, inv_sqrt_c, inv_sqrt_ca):
    total = p_ref[0] + p_ref[1]
    node = jnp.dot(total, wpost_ref[...],
                   preferred_element_type=jnp.float32) * inv_sqrt_c
    attr = attr_ref[...]
    a1 = jnp.dot(attr, wc_ref[0], preferred_element_type=jnp.float32)
    a2 = jnp.dot(attr, wc_ref[1], preferred_element_type=jnp.float32)
    a3 = jnp.dot(attr, wc_ref[2], preferred_element_type=jnp.float32)
    node2 = node * node
    poly = a1 * node + a2 * node2 + a3 * node2 * node
    nf = nf_ref[...]
    upd = jnp.zeros_like(poly)
    for a in range(a_dim):
        upd = upd + jnp.dot(nf * attr[:, a][:, None], wres_ref[a],
                            preferred_element_type=jnp.float32)
    contracted = poly + upd * inv_sqrt_ca
    out_ref[...] = jnp.dot(contracted, wout_ref[...],
                           preferred_element_type=jnp.float32) * inv_sqrt_c


def _post(partials, nf, attr, w_post, wc_stack, wres_t, w_out):
    n, c = nf.shape
    a_dim = attr.shape[1]
    bn = 1000
    return pl.pallas_call(
        functools.partial(_post_body, a_dim=a_dim, c=c,
                          inv_sqrt_c=1.0 / math.sqrt(c),
                          inv_sqrt_ca=1.0 / math.sqrt(c * a_dim)),
        grid=(n // bn,),
        in_specs=[
            pl.BlockSpec((NC, bn, c), lambda i: (_z(), i, _z())),
            pl.BlockSpec((bn, c), lambda i: (i, _z())),
            pl.BlockSpec((bn, a_dim), lambda i: (i, _z())),
            pl.BlockSpec((c, c), lambda i: (_z(), _z())),
            pl.BlockSpec((3, a_dim, c), lambda i: (_z(), _z(), _z())),
            pl.BlockSpec((a_dim, c, c), lambda i: (_z(), _z(), _z())),
            pl.BlockSpec((c, c), lambda i: (_z(), _z())),
        ],
        out_specs=pl.BlockSpec((bn, c), lambda i: (i, _z())),
        out_shape=jax.ShapeDtypeStruct((n, c), jnp.float32),
    )(partials, nf, attr, w_post, wc_stack, wres_t, w_out)


def kernel(node_features, node_attributes, sph_harmonics, radial_basis,
           edge_index, W_pre, W1, W2, W3, W_post, W_c1, W_c2, W_c3,
           W_res, W_out):
    n, c = node_features.shape
    e = edge_index.shape[1]
    k = 64
    grp = 8
    e_pad = -(-e // (NW * k * grp)) * (NW * k * grp)
    pad = e_pad - e

    h = _pre_linear(node_features, W_pre)
    rb_p = jnp.pad(radial_basis, ((0, pad), (0, 0))) if pad else radial_basis
    sph_p = jnp.pad(sph_harmonics, ((0, pad), (0, 0))) if pad else sph_harmonics
    coeff = _edge_coeff(rb_p, sph_p, W1, W2, W3)

    src = edge_index[0].astype(jnp.int32)
    dst = edge_index[1].astype(jnp.int32)
    if pad:
        src = jnp.pad(src, (0, pad))
        dst = jnp.pad(dst, (0, pad))
    src3d = src.reshape(NW, e_pad // (NW * k), k)
    dst3d = dst.reshape(NW, e_pad // (NW * k), k)
    partials = _sc_aggregate(h, coeff, src3d, dst3d, n=n, c=c,
                             e_pad=e_pad, k=k, grp=grp)
    partials = partials[:, :n, :]

    wc_stack = jnp.stack([W_c1, W_c2, W_c3])
    wres_t = jnp.transpose(W_res, (1, 0, 2))
    return _post(partials, node_features, node_attributes, W_post,
                 wc_stack, wres_t, W_out)
